# Initial kernel scaffold; baseline (speedup 1.0000x reference)
import jax
import jax.numpy as jnp
from jax.experimental import pallas as pl


def kernel(x, edge_index, W1, b1, W2, b2):
    raise NotImplementedError("write your pallas kernel here")


# speedup vs baseline: 1.7593x; 1.7593x over previous
import functools

import jax
import jax.numpy as jnp
from jax import lax
from jax.experimental import pallas as pl
from jax.experimental.pallas import tpu as pltpu
from jax.experimental.pallas import tpu_sc as plsc

N_NODES = 10000
N_EDGES = 160000
D_FEAT = 256
H1 = 1024
H2 = 512

_NC = 2
_NS = 16
_NW = _NC * _NS

_ROWS_PER_W = (2 * N_EDGES) // _NW
_CHUNK = 200
_NCHUNK = _ROWS_PER_W // _CHUNK


def _sc_gather_body(x_hbm, idx_hbm, out_hbm, idx_v, rows_v, sem):
    wid = lax.axis_index("s") * _NC + lax.axis_index("c")
    base = wid * _ROWS_PER_W
    pltpu.sync_copy(idx_hbm.at[pl.ds(base, _ROWS_PER_W)], idx_v)

    def chunk(c, carry):
        off = c * _CHUNK
        pltpu.async_copy(x_hbm.at[idx_v.at[pl.ds(off, _CHUNK)]], rows_v, sem).wait()
        pltpu.sync_copy(rows_v, out_hbm.at[pl.ds(base + off, _CHUNK)])
        return carry

    lax.fori_loop(0, _NCHUNK, chunk, 0)


_sc_gather = functools.partial(
    pl.kernel,
    mesh=plsc.VectorSubcoreMesh(core_axis_name="c", subcore_axis_name="s"),
    out_type=jax.ShapeDtypeStruct((2 * N_EDGES, D_FEAT), jnp.float32),
    scratch_types=[
        pltpu.VMEM((_ROWS_PER_W,), jnp.int32),
        pltpu.VMEM((_CHUNK, D_FEAT), jnp.float32),
        pltpu.SemaphoreType.DMA,
    ],
)(_sc_gather_body)


_BE = 640
_NB = N_EDGES // _BE


def _mlp_body(src_ref, dst_ref, w1a_ref, w1b_ref, b1_ref, w2_ref, b2_ref, out_ref):
    src = src_ref[...].astype(jnp.bfloat16)
    dst = dst_ref[...].astype(jnp.bfloat16)
    h = jnp.dot(src, w1a_ref[...], preferred_element_type=jnp.float32)
    h = h + jnp.dot(dst, w1b_ref[...], preferred_element_type=jnp.float32)
    h = jnp.maximum(h + b1_ref[...], 0.0).astype(jnp.bfloat16)
    o = jnp.dot(h, w2_ref[...], preferred_element_type=jnp.float32)
    out_ref[...] = jnp.maximum(o + b2_ref[...], 0.0)


_mlp = pl.pallas_call(
    _mlp_body,
    grid=(_NB,),
    in_specs=[
        pl.BlockSpec((_BE, D_FEAT), lambda i: (i, 0)),
        pl.BlockSpec((_BE, D_FEAT), lambda i: (i + _NB, 0)),
        pl.BlockSpec((D_FEAT, H1), lambda i: (0, 0)),
        pl.BlockSpec((D_FEAT, H1), lambda i: (0, 0)),
        pl.BlockSpec((1, H1), lambda i: (0, 0)),
        pl.BlockSpec((H1, H2), lambda i: (0, 0)),
        pl.BlockSpec((1, H2), lambda i: (0, 0)),
    ],
    out_specs=pl.BlockSpec((_BE, H2), lambda i: (i, 0)),
    out_shape=jax.ShapeDtypeStruct((N_EDGES, H2), jnp.float32),
    compiler_params=pltpu.CompilerParams(
        dimension_semantics=("arbitrary",),
    ),
)


def kernel(x, edge_index, W1, b1, W2, b2):
    idx = edge_index.reshape(-1)
    gath = _sc_gather(x, idx)
    w1a = W1[:D_FEAT].astype(jnp.bfloat16)
    w1b = W1[D_FEAT:].astype(jnp.bfloat16)
    w2 = W2.astype(jnp.bfloat16)
    return _mlp(gath, gath, w1a, w1b, b1.reshape(1, H1), w2, b2.reshape(1, H2))


# speedup vs baseline: 2.0500x; 1.1652x over previous
import functools

import jax
import jax.numpy as jnp
from jax import lax
from jax.experimental import pallas as pl
from jax.experimental.pallas import tpu as pltpu
from jax.experimental.pallas import tpu_sc as plsc

N_NODES = 10000
N_EDGES = 160000
D_FEAT = 256
H1 = 1024
H2 = 512

_NSEG = 5
_ESEG = N_EDGES // _NSEG

_NC = 2
_NS = 16
_NW = _NC * _NS

_ROWS_PER_W = (2 * _ESEG) // _NW
_CHUNK = 200
_NCHUNK = _ROWS_PER_W // _CHUNK


def _sc_gather_body(x_hbm, idx_hbm, out_hbm, idx_v, rows_v, sem):
    wid = lax.axis_index("s") * _NC + lax.axis_index("c")
    base = wid * _ROWS_PER_W
    pltpu.sync_copy(idx_hbm.at[pl.ds(base, _ROWS_PER_W)], idx_v)

    def chunk(c, carry):
        off = c * _CHUNK
        pltpu.async_copy(x_hbm.at[idx_v.at[pl.ds(off, _CHUNK)]], rows_v, sem).wait()
        pltpu.sync_copy(rows_v, out_hbm.at[pl.ds(base + off, _CHUNK)])
        return carry

    lax.fori_loop(0, _NCHUNK, chunk, 0)


_sc_gather = functools.partial(
    pl.kernel,
    mesh=plsc.VectorSubcoreMesh(core_axis_name="c", subcore_axis_name="s"),
    out_type=jax.ShapeDtypeStruct((2 * _ESEG, D_FEAT), jnp.float32),
    scratch_types=[
        pltpu.VMEM((_ROWS_PER_W,), jnp.int32),
        pltpu.VMEM((_CHUNK, D_FEAT), jnp.float32),
        pltpu.SemaphoreType.DMA,
    ],
)(_sc_gather_body)


_BE = 640
_NB = _ESEG // _BE


def _mlp_body(src_ref, dst_ref, w1a_ref, w1b_ref, b1_ref, w2_ref, b2_ref, out_ref):
    src = src_ref[...].astype(jnp.bfloat16)
    dst = dst_ref[...].astype(jnp.bfloat16)
    h = jnp.dot(src, w1a_ref[...], preferred_element_type=jnp.float32)
    h = h + jnp.dot(dst, w1b_ref[...], preferred_element_type=jnp.float32)
    h = jnp.maximum(h + b1_ref[...], 0.0).astype(jnp.bfloat16)
    o = jnp.dot(h, w2_ref[...], preferred_element_type=jnp.float32)
    out_ref[...] = jnp.maximum(o + b2_ref[...], 0.0)


def _mlp_body_acc(src_ref, dst_ref, w1a_ref, w1b_ref, b1_ref, w2_ref, b2_ref,
                  acc_ref, out_ref):
    del acc_ref
    _mlp_body(src_ref, dst_ref, w1a_ref, w1b_ref, b1_ref, w2_ref, b2_ref, out_ref)


def _mk_mlp(seg, acc):
    in_specs = [
        pl.BlockSpec((_BE, D_FEAT), lambda i: (i, 0)),
        pl.BlockSpec((_BE, D_FEAT), lambda i: (i + _NB, 0)),
        pl.BlockSpec((D_FEAT, H1), lambda i: (0, 0)),
        pl.BlockSpec((D_FEAT, H1), lambda i: (0, 0)),
        pl.BlockSpec((1, H1), lambda i: (0, 0)),
        pl.BlockSpec((H1, H2), lambda i: (0, 0)),
        pl.BlockSpec((1, H2), lambda i: (0, 0)),
    ]
    kwargs = {}
    if acc:
        in_specs.append(pl.BlockSpec(memory_space=pl.ANY))
        kwargs["input_output_aliases"] = {7: 0}
    return pl.pallas_call(
        _mlp_body_acc if acc else _mlp_body,
        grid=(_NB,),
        in_specs=in_specs,
        out_specs=pl.BlockSpec((_BE, H2), lambda i, seg=seg: (i + seg * _NB, 0)),
        out_shape=jax.ShapeDtypeStruct((N_EDGES, H2), jnp.float32),
        compiler_params=pltpu.CompilerParams(
            dimension_semantics=("arbitrary",),
        ),
        **kwargs,
    )


_mlps = [_mk_mlp(s, s > 0) for s in range(_NSEG)]


def kernel(x, edge_index, W1, b1, W2, b2):
    w1a = W1[:D_FEAT].astype(jnp.bfloat16)
    w1b = W1[D_FEAT:].astype(jnp.bfloat16)
    w2 = W2.astype(jnp.bfloat16)
    b1r = b1.reshape(1, H1)
    b2r = b2.reshape(1, H2)
    idx_seg = edge_index.reshape(2, _NSEG, _ESEG)
    gaths = [_sc_gather(x, idx_seg[:, s, :].reshape(-1)) for s in range(_NSEG)]
    out = _mlps[0](gaths[0], gaths[0], w1a, w1b, b1r, w2, b2r)
    for s in range(1, _NSEG):
        out = _mlps[s](gaths[s], gaths[s], w1a, w1b, b1r, w2, b2r, out)
    return out
